# baseline (speedup 1.0000x reference)
import functools

import jax
import jax.numpy as jnp
from jax import lax
from jax.experimental import pallas as pl
from jax.experimental.pallas import tpu as pltpu
from jax.experimental.pallas import tpu_sc as plsc

_NC = 2
_NS = 16
_NW = _NC * _NS
_L = 16
_SG = 8
_BG = _NW // _SG


def _sc_body(row_hbm, col_hbm, out_hbm, row_v, col_v, chunk_v, sem, sem2,
             *, b, h, w, d):
    gr = d // _L
    ti = h // _SG
    bn = b // _BG

    cid = lax.axis_index("c")
    sid = lax.axis_index("s")
    wid = sid * _NC + cid
    sg = wid % _SG
    bg = wid // _SG
    i0 = sg * ti
    b0 = bg * bn

    stage_row = pltpu.async_copy(row_hbm.at[pl.ds(0, h)], row_v, sem2)
    stage_col = pltpu.async_copy(col_hbm.at[pl.ds(0, w)], col_v, sem2)
    stage_row.wait()
    stage_col.wait()

    descs = []
    for r in range(ti):
        rvals = [row_v[i0 + r, pl.ds(cg * _L, _L)] for cg in range(gr)]

        def body(j, _, r=r, rvals=rvals):
            for cg in range(gr):
                chunk_v[r, j, pl.ds(cg * _L, _L)] = (
                    col_v[j, pl.ds(cg * _L, _L)])
            for cg in range(gr):
                chunk_v[r, j, pl.ds(d + cg * _L, _L)] = rvals[cg]
            return 0

        if r == 0:
            hw = 8
            lax.fori_loop(0, hw, body, 0)
            descs.extend(
                pltpu.async_copy(chunk_v.at[0, pl.ds(0, hw)],
                                 out_hbm.at[b0 + k, i0, pl.ds(0, hw)], sem)
                for k in range(bn))
            lax.fori_loop(hw, w, body, 0)
            descs.extend(
                pltpu.async_copy(chunk_v.at[0, pl.ds(hw, w - hw)],
                                 out_hbm.at[b0 + k, i0, pl.ds(hw, w - hw)],
                                 sem)
                for k in range(bn))
            continue
        lax.fori_loop(0, w, body, 0)
        descs.extend(
            pltpu.async_copy(chunk_v.at[r], out_hbm.at[b0 + k, i0 + r], sem)
            for k in range(bn))
    for dsc in descs:
        dsc.wait()


def kernel(x, row_weight, col_weight):
    b = x.shape[0]
    h, w = x.shape[-2], x.shape[-1]
    d = row_weight.shape[1]

    mesh = plsc.VectorSubcoreMesh(core_axis_name="c", subcore_axis_name="s")
    body = functools.partial(_sc_body, b=b, h=h, w=w, d=d)
    run = pl.kernel(
        body,
        out_type=jax.ShapeDtypeStruct((b, h, w, 2 * d), jnp.float32),
        mesh=mesh,
        scratch_types=[
            pltpu.VMEM((h, d), jnp.float32),
            pltpu.VMEM((w, d), jnp.float32),
            pltpu.VMEM((h // _SG, w, 2 * d), jnp.float32),
            pltpu.SemaphoreType.DMA,
            pltpu.SemaphoreType.DMA,
        ],
        compiler_params=pltpu.CompilerParams(needs_layout_passes=False),
    )
    out = run(row_weight, col_weight)
    return out.transpose(0, 3, 1, 2)


# speedup vs baseline: 1.0505x; 1.0505x over previous
import functools

import jax
import jax.numpy as jnp
from jax import lax
from jax.experimental import pallas as pl
from jax.experimental.pallas import tpu as pltpu
from jax.experimental.pallas import tpu_sc as plsc

_NC = 2
_NS = 16
_NW = _NC * _NS
_L = 16
_SG = 8
_BG = _NW // _SG


def _sc_body(row_hbm, col_hbm, out_hbm, row_v, col_v, chunk_v, sem, sem2,
             *, b, h, w, d):
    gr = d // _L
    ti = h // _SG
    bn = b // _BG

    cid = lax.axis_index("c")
    sid = lax.axis_index("s")
    wid = sid * _NC + cid
    sg = wid % _SG
    bg = wid // _SG
    i0 = sg * ti
    b0 = bg * bn

    stage_row = pltpu.async_copy(row_hbm.at[pl.ds(0, h)], row_v, sem2)
    stage_col = pltpu.async_copy(col_hbm.at[pl.ds(0, w)], col_v, sem2)
    stage_row.wait()
    stage_col.wait()

    descs = []
    for r in range(ti):
        rvals = [row_v[i0 + r, pl.ds(cg * _L, _L)] for cg in range(gr)]

        def body(j, _, r=r, rvals=rvals):
            for cg in range(gr):
                chunk_v[r, j, pl.ds(cg * _L, _L)] = (
                    col_v[j, pl.ds(cg * _L, _L)])
            for cg in range(gr):
                chunk_v[r, j, pl.ds(d + cg * _L, _L)] = rvals[cg]
            return 0

        lax.fori_loop(0, w, body, 0)
        descs.extend(
            pltpu.async_copy(chunk_v.at[r], out_hbm.at[b0 + k, i0 + r], sem)
            for k in range(bn))
    for dsc in descs:
        dsc.wait()


def kernel(x, row_weight, col_weight):
    b = x.shape[0]
    h, w = x.shape[-2], x.shape[-1]
    d = row_weight.shape[1]

    mesh = plsc.VectorSubcoreMesh(core_axis_name="c", subcore_axis_name="s")
    body = functools.partial(_sc_body, b=b, h=h, w=w, d=d)
    run = pl.kernel(
        body,
        out_type=jax.ShapeDtypeStruct((b, h, w, 2 * d), jnp.float32),
        mesh=mesh,
        scratch_types=[
            pltpu.VMEM((h, d), jnp.float32),
            pltpu.VMEM((w, d), jnp.float32),
            pltpu.VMEM((h // _SG, w, 2 * d), jnp.float32),
            pltpu.SemaphoreType.DMA,
            pltpu.SemaphoreType.DMA,
        ],
        compiler_params=pltpu.CompilerParams(needs_layout_passes=False),
    )
    out = run(row_weight, col_weight)
    return out.transpose(0, 3, 1, 2)
